# baseline (speedup 1.0000x reference)
import functools

import jax
import jax.numpy as jnp
from jax import lax
from jax.experimental import pallas as pl
from jax.experimental.pallas import tpu as pltpu
from jax.experimental.pallas import tpu_sc as plsc

N_NODES = 100000
N_EDGES = 3200000
LINK_LENGTH = 100.0
GRAVITY = 9.81
WATER_DENSITY = 1000.0
ICE_DENSITY = 917.0
LATENT_HEAT = 334000.0
WATER_VISCOSITY = 1.787e-06
ICE_FLUIDITY = 6e-24
TILL_FRICTION = 0.5
FLOW_REGIME_SCALAR = 0.001
N_FP_ITERS = 15

NC, NS, L = 2, 16, 16
NW = NC * NS
EPW = N_EDGES // NW
CHUNK = 4000
NCHUNKS = EPW // CHUNK
UNROLL = 5

E_SPLIT = 1664000

_MESH = plsc.VectorSubcoreMesh(
    core_axis_name="c", subcore_axis_name="s", num_cores=NC, num_subcores=NS)
_SC_PARAMS = pltpu.CompilerParams(
    needs_layout_passes=False, use_tc_tiling_on_sc=False)

NR, NCL = 100, 1000
ECL = 128


def _worker_id():
    return lax.axis_index("s") * NC + lax.axis_index("c")


def _make_gather(mode, estart, ecount):
    epw = ecount // NW
    nchunks = epw // CHUNK

    def body(tab_hbm, src_hbm, dst_hbm, out_hbm, table, shared,
             srcv0, dstv0, outv0, srcv1, dstv1, outv1,
             tsem, isem0, isem1, osem0, osem1):
        sid = lax.axis_index("s")
        obase = _worker_id() * epw
        base = estart + obase
        bufs = ((srcv0, dstv0, outv0, isem0, osem0),
                (srcv1, dstv1, outv1, isem1, osem1))

        def start_in(ci):
            s, d, _, isem, _ = bufs[ci % 2]
            off = base + ci * CHUNK
            c1 = pltpu.async_copy(src_hbm.at[pl.ds(off, CHUNK)], s, isem)
            c2 = pltpu.async_copy(dst_hbm.at[pl.ds(off, CHUNK)], d, isem)
            return (c1, c2)

        in_cp = {0: start_in(0), 1: start_in(1)}

        @pl.when(sid == 0)
        def _():
            pltpu.sync_copy(tab_hbm, shared)

        plsc.subcore_barrier()
        table_cp = pltpu.async_copy(shared, table, tsem)

        out_cp = {}
        for ci in range(nchunks):
            s, d, o, isem, osem = bufs[ci % 2]
            if ci + 1 < nchunks and ci > 0:
                in_cp[ci + 1] = start_in(ci + 1)
            for cp in in_cp.pop(ci):
                cp.wait()
            if ci == 0:
                table_cp.wait()
            if ci >= 2:
                out_cp.pop(ci - 2).wait()

            @plsc.parallel_loop(0, CHUNK, step=L, unroll=UNROLL)
            def _(i, _s=s, _d=d, _o=o):
                sv = _s[pl.ds(i, L)]
                dv = _d[pl.ds(i, L)]
                ts = plsc.load_gather(table, [sv])
                td = plsc.load_gather(table, [dv])
                if mode == 0:
                    _o[pl.ds(i, L)] = (td - ts) / LINK_LENGTH
                else:
                    _o[pl.ds(i, L)] = 0.5 * (ts + td)

            out_cp[ci] = pltpu.async_copy(
                o, out_hbm.at[pl.ds(obase + ci * CHUNK, CHUNK)], osem)
        for cp in out_cp.values():
            cp.wait()

    return pl.kernel(
        body,
        out_type=jax.ShapeDtypeStruct((ecount,), jnp.float32),
        mesh=_MESH,
        compiler_params=_SC_PARAMS,
        scratch_types=[
            pltpu.VMEM((N_NODES,), jnp.float32),
            pltpu.VMEM_SHARED((N_NODES,), jnp.float32),
            pltpu.VMEM((CHUNK,), jnp.int32),
            pltpu.VMEM((CHUNK,), jnp.int32),
            pltpu.VMEM((CHUNK,), jnp.float32),
            pltpu.VMEM((CHUNK,), jnp.int32),
            pltpu.VMEM((CHUNK,), jnp.int32),
            pltpu.VMEM((CHUNK,), jnp.float32),
            pltpu.SemaphoreType.DMA,
            pltpu.SemaphoreType.DMA,
            pltpu.SemaphoreType.DMA,
            pltpu.SemaphoreType.DMA,
            pltpu.SemaphoreType.DMA,
        ],
    )


_gather_grad = _make_gather(0, 0, N_EDGES)
_gather_full = _make_gather(1, 0, N_EDGES)
_gather_mean_a = _make_gather(1, 0, E_SPLIT)
_gather_mean_b = _make_gather(1, E_SPLIT, N_EDGES - E_SPLIT)


def _scatter_body(src_hbm, dst_hbm, val_hbm, velp_hbm, degp_hbm, table,
                  srcv0, dstv0, valv0, srcv1, dstv1, valv1, isem0, isem1):
    wid = _worker_id()
    base = wid * EPW
    bufs = ((srcv0, dstv0, valv0, isem0),
            (srcv1, dstv1, valv1, isem1))

    def start_in(ci, with_vals):
        s, d, v, isem = bufs[ci % 2]
        off = base + ci * CHUNK
        cps = [pltpu.async_copy(src_hbm.at[pl.ds(off, CHUNK)], s, isem),
               pltpu.async_copy(dst_hbm.at[pl.ds(off, CHUNK)], d, isem)]
        if with_vals:
            cps.append(
                pltpu.async_copy(val_hbm.at[pl.ds(off, CHUNK)], v, isem))
        return cps

    for phase, out_hbm in ((0, velp_hbm), (1, degp_hbm)):
        with_vals = phase == 0
        in_cp = {0: start_in(0, with_vals)}

        @plsc.parallel_loop(0, N_NODES, step=L, unroll=25)
        def _(i):
            table[pl.ds(i, L)] = jnp.zeros((L,), jnp.float32)

        for ci in range(NCHUNKS):
            s, d, v, isem = bufs[ci % 2]
            if ci + 1 < NCHUNKS:
                in_cp[ci + 1] = start_in(ci + 1, with_vals)
            for cp in in_cp.pop(ci):
                cp.wait()

            @plsc.parallel_loop(0, CHUNK, step=L, unroll=UNROLL)
            def _(i, _s=s, _d=d, _v=v, _wv=with_vals):
                sv = _s[pl.ds(i, L)]
                dv = _d[pl.ds(i, L)]
                if _wv:
                    vv = _v[pl.ds(i, L)]
                else:
                    vv = jnp.ones((L,), jnp.float32)
                plsc.addupdate_scatter(table, [sv], vv)
                plsc.addupdate_scatter(table, [dv], vv)

        pltpu.sync_copy(table, out_hbm.at[wid])


_scatter_both = pl.kernel(
    _scatter_body,
    out_type=(jax.ShapeDtypeStruct((NW, N_NODES), jnp.float32),
              jax.ShapeDtypeStruct((NW, N_NODES), jnp.float32)),
    mesh=_MESH,
    compiler_params=_SC_PARAMS,
    scratch_types=[
        pltpu.VMEM((N_NODES,), jnp.float32),
        pltpu.VMEM((CHUNK,), jnp.int32),
        pltpu.VMEM((CHUNK,), jnp.int32),
        pltpu.VMEM((CHUNK,), jnp.float32),
        pltpu.VMEM((CHUNK,), jnp.int32),
        pltpu.VMEM((CHUNK,), jnp.int32),
        pltpu.VMEM((CHUNK,), jnp.float32),
        pltpu.SemaphoreType.DMA,
        pltpu.SemaphoreType.DMA,
    ],
)


def _node1_body(head_ref, bed_ref, ovb_ref, bnd_ref, head_o, neff_o):
    h = head_ref[...]
    b = bed_ref[...]
    ov = ovb_ref[...]
    h = jnp.where(bnd_ref[...] != 0.0, b, h)
    head_o[...] = h
    wp = WATER_DENSITY * GRAVITY * (h - b)
    ne = ov - wp
    ne = jnp.where(ne > ov, ov, ne)
    ne = jnp.where(ne < 10000.0, 10000.0, ne)
    neff_o[...] = ne


def _node1(head2, bed2, ovb2, bnd2):
    return pl.pallas_call(
        _node1_body,
        out_shape=(
            jax.ShapeDtypeStruct((NR, NCL), jnp.float32),
            jax.ShapeDtypeStruct((NR, NCL), jnp.float32),
        ),
    )(head2, bed2, ovb2, bnd2)


def _node2_body(velp_ref, degp_ref, neff_ref, geo_ref, melt_o, cond_o):
    vs = jnp.sum(velp_ref[...], axis=0)
    dg = jnp.sum(degp_ref[...], axis=0)
    sliding = vs / jnp.maximum(dg, 1.0)
    ne = neff_ref[...]
    shear = TILL_FRICTION * ne
    friction = jnp.abs(sliding * shear)
    melt = (geo_ref[...] + friction) / LATENT_HEAT
    melt_o[...] = melt
    cond_o[...] = melt / ICE_DENSITY / (ICE_FLUIDITY * (ne * ne * ne))


def _node2(velp, degp, neff2, geo2):
    return pl.pallas_call(
        _node2_body,
        out_shape=(
            jax.ShapeDtypeStruct((NR, NCL), jnp.float32),
            jax.ShapeDtypeStruct((NR, NCL), jnp.float32),
        ),
    )(velp, degp, neff2, geo2)


def _fp_body(cal_ref, grad_ref, re_ref, re_o, tr_o, di_o):
    c = cal_ref[...]
    num = c * c * c * GRAVITY
    g = grad_ref[...]
    r = re_ref[...]
    k = jnp.abs(num * g) * (1.0 / (24.0 * WATER_VISCOSITY * WATER_VISCOSITY))
    for _ in range(N_FP_ITERS):
        r = 0.5 * r + k / (1.0 + FLOW_REGIME_SCALAR * r)
    t = num / (12.0 * WATER_VISCOSITY * (1.0 + FLOW_REGIME_SCALAR * r))
    re_o[...] = r
    tr_o[...] = t
    di_o[...] = -t * g


_EROWS = N_EDGES // ECL
_ROWS_A = E_SPLIT // ECL
_BROW = 1000


def _fp_full(cal, grad, re):
    bspec = pl.BlockSpec((_BROW, ECL), lambda i: (i, 0))
    shape = jax.ShapeDtypeStruct((_EROWS, ECL), jnp.float32)
    return pl.pallas_call(
        _fp_body,
        grid=(_EROWS // _BROW,),
        in_specs=[bspec, bspec, bspec],
        out_specs=(bspec, bspec, bspec),
        out_shape=(shape, shape, shape),
    )(cal.reshape(_EROWS, ECL), grad, re)


def _fp_a(cal_a, grad, re):
    bspec = pl.BlockSpec((_BROW, ECL), lambda i: (i, 0))
    shape = jax.ShapeDtypeStruct((_EROWS, ECL), jnp.float32)
    return pl.pallas_call(
        _fp_body,
        grid=(_ROWS_A // _BROW,),
        in_specs=[bspec, bspec, bspec],
        out_specs=(bspec, bspec, bspec),
        out_shape=(shape, shape, shape),
    )(cal_a.reshape(_ROWS_A, ECL), grad, re)


def _fp_b_body(cal_ref, grad_ref, re_ref, _a, _b, _c, re_o, tr_o, di_o):
    _fp_body(cal_ref, grad_ref, re_ref, re_o, tr_o, di_o)


def _fp_b(cal_b, grad, re, re_f, tr_f, di_f):
    rows_b = _EROWS - _ROWS_A
    near = pl.BlockSpec((_BROW, ECL), lambda i: (i, 0))
    far = pl.BlockSpec((_BROW, ECL), lambda i: (i + _ROWS_A // _BROW, 0))
    anyspec = pl.BlockSpec(memory_space=pl.ANY)
    shape = jax.ShapeDtypeStruct((_EROWS, ECL), jnp.float32)
    return pl.pallas_call(
        _fp_b_body,
        grid=(rows_b // _BROW,),
        in_specs=[near, far, far, anyspec, anyspec, anyspec],
        out_specs=(far, far, far),
        out_shape=(shape, shape, shape),
        input_output_aliases={3: 0, 4: 1, 5: 2},
    )(cal_b.reshape(rows_b, ECL), grad, re, re_f, tr_f, di_f)


def kernel(head, Re, edge_index, bedrock_elevation, overburden_pressure,
           geothermal_heat_flux, ice_sliding_velocity, node_is_boundary):
    src = edge_index[0]
    dst = edge_index[1]
    bnd2 = node_is_boundary.astype(jnp.float32).reshape(NR, NCL)

    velp, degp = _scatter_both(src, dst, ice_sliding_velocity)

    head_p2, neff2 = _node1(
        head.reshape(NR, NCL),
        bedrock_elevation.reshape(NR, NCL),
        overburden_pressure.reshape(NR, NCL),
        bnd2,
    )
    head_p = head_p2.reshape(-1)

    grad = _gather_grad(head_p, src, dst)

    melt2, cond2 = _node2(
        velp.reshape(NW, NR, NCL),
        degp.reshape(NW, NR, NCL),
        neff2,
        geothermal_heat_flux.reshape(NR, NCL),
    )
    cond = cond2.reshape(-1)

    grad2 = grad.reshape(_EROWS, ECL)
    re2 = Re.reshape(_EROWS, ECL)
    cal = _gather_full(cond, src, dst)
    re_o, tr_o, di_o = _fp_full(cal, grad2, re2)

    return (
        head_p,
        grad,
        neff2.reshape(-1),
        melt2.reshape(-1),
        cond,
        re_o.reshape(-1),
        tr_o.reshape(-1),
        di_o.reshape(-1),
    )


# speedup vs baseline: 1.0234x; 1.0234x over previous
import functools

import jax
import jax.numpy as jnp
from jax import lax
from jax.experimental import pallas as pl
from jax.experimental.pallas import tpu as pltpu
from jax.experimental.pallas import tpu_sc as plsc

N_NODES = 100000
N_EDGES = 3200000
LINK_LENGTH = 100.0
GRAVITY = 9.81
WATER_DENSITY = 1000.0
ICE_DENSITY = 917.0
LATENT_HEAT = 334000.0
WATER_VISCOSITY = 1.787e-06
ICE_FLUIDITY = 6e-24
TILL_FRICTION = 0.5
FLOW_REGIME_SCALAR = 0.001
N_FP_ITERS = 15

NC, NS, L = 2, 16, 16
NW = NC * NS
EPW = N_EDGES // NW
CHUNK = 4000
NCHUNKS = EPW // CHUNK
UNROLL = 5

E_SPLIT = 1664000

_MESH = plsc.VectorSubcoreMesh(
    core_axis_name="c", subcore_axis_name="s", num_cores=NC, num_subcores=NS)
_SC_PARAMS = pltpu.CompilerParams(
    needs_layout_passes=False, use_tc_tiling_on_sc=False)

NR, NCL = 100, 1000
ECL = 128


def _worker_id():
    return lax.axis_index("s") * NC + lax.axis_index("c")


def _make_gather(mode, estart, ecount):
    epw = ecount // NW
    nchunks = epw // CHUNK

    def body(tab_hbm, src_hbm, dst_hbm, out_hbm, table, shared,
             srcv0, dstv0, outv0, srcv1, dstv1, outv1,
             tsem, isem0, isem1, osem0, osem1):
        sid = lax.axis_index("s")
        obase = _worker_id() * epw
        base = estart + obase
        bufs = ((srcv0, dstv0, outv0, isem0, osem0),
                (srcv1, dstv1, outv1, isem1, osem1))

        def start_in(ci):
            s, d, _, isem, _ = bufs[ci % 2]
            off = base + ci * CHUNK
            c1 = pltpu.async_copy(src_hbm.at[pl.ds(off, CHUNK)], s, isem)
            c2 = pltpu.async_copy(dst_hbm.at[pl.ds(off, CHUNK)], d, isem)
            return (c1, c2)

        in_cp = {0: start_in(0), 1: start_in(1)}

        @pl.when(sid == 0)
        def _():
            pltpu.sync_copy(tab_hbm, shared)

        plsc.subcore_barrier()
        table_cp = pltpu.async_copy(shared, table, tsem)

        out_cp = {}
        for ci in range(nchunks):
            s, d, o, isem, osem = bufs[ci % 2]
            if ci + 1 < nchunks and ci > 0:
                in_cp[ci + 1] = start_in(ci + 1)
            for cp in in_cp.pop(ci):
                cp.wait()
            if ci == 0:
                table_cp.wait()
            if ci >= 2:
                out_cp.pop(ci - 2).wait()

            @plsc.parallel_loop(0, CHUNK, step=L, unroll=UNROLL)
            def _(i, _s=s, _d=d, _o=o):
                sv = _s[pl.ds(i, L)]
                dv = _d[pl.ds(i, L)]
                ts = plsc.load_gather(table, [sv])
                td = plsc.load_gather(table, [dv])
                if mode == 0:
                    _o[pl.ds(i, L)] = (td - ts) / LINK_LENGTH
                else:
                    _o[pl.ds(i, L)] = 0.5 * (ts + td)

            out_cp[ci] = pltpu.async_copy(
                o, out_hbm.at[pl.ds(obase + ci * CHUNK, CHUNK)], osem)
        for cp in out_cp.values():
            cp.wait()

    return pl.kernel(
        body,
        out_type=jax.ShapeDtypeStruct((ecount,), jnp.float32),
        mesh=_MESH,
        compiler_params=_SC_PARAMS,
        scratch_types=[
            pltpu.VMEM((N_NODES,), jnp.float32),
            pltpu.VMEM_SHARED((N_NODES,), jnp.float32),
            pltpu.VMEM((CHUNK,), jnp.int32),
            pltpu.VMEM((CHUNK,), jnp.int32),
            pltpu.VMEM((CHUNK,), jnp.float32),
            pltpu.VMEM((CHUNK,), jnp.int32),
            pltpu.VMEM((CHUNK,), jnp.int32),
            pltpu.VMEM((CHUNK,), jnp.float32),
            pltpu.SemaphoreType.DMA,
            pltpu.SemaphoreType.DMA,
            pltpu.SemaphoreType.DMA,
            pltpu.SemaphoreType.DMA,
            pltpu.SemaphoreType.DMA,
        ],
    )


_gather_grad = _make_gather(0, 0, N_EDGES)
_gather_full = _make_gather(1, 0, N_EDGES)
_gather_mean_a = _make_gather(1, 0, E_SPLIT)
_gather_mean_b = _make_gather(1, E_SPLIT, N_EDGES - E_SPLIT)


def _scatter_body(src_hbm, dst_hbm, val_hbm, velp_hbm, degp_hbm, table,
                  srcv0, dstv0, valv0, srcv1, dstv1, valv1, isem0, isem1):
    wid = _worker_id()
    base = wid * EPW
    bufs = ((srcv0, dstv0, valv0, isem0),
            (srcv1, dstv1, valv1, isem1))

    def start_in(ci, with_vals):
        s, d, v, isem = bufs[ci % 2]
        off = base + ci * CHUNK
        cps = [pltpu.async_copy(src_hbm.at[pl.ds(off, CHUNK)], s, isem),
               pltpu.async_copy(dst_hbm.at[pl.ds(off, CHUNK)], d, isem)]
        if with_vals:
            cps.append(
                pltpu.async_copy(val_hbm.at[pl.ds(off, CHUNK)], v, isem))
        return cps

    for phase, out_hbm in ((0, velp_hbm), (1, degp_hbm)):
        with_vals = phase == 0
        in_cp = {0: start_in(0, with_vals)}

        @plsc.parallel_loop(0, N_NODES, step=L, unroll=25)
        def _(i):
            table[pl.ds(i, L)] = jnp.zeros((L,), jnp.float32)

        for ci in range(NCHUNKS):
            s, d, v, isem = bufs[ci % 2]
            if ci + 1 < NCHUNKS:
                in_cp[ci + 1] = start_in(ci + 1, with_vals)
            for cp in in_cp.pop(ci):
                cp.wait()

            @plsc.parallel_loop(0, CHUNK, step=L, unroll=UNROLL)
            def _(i, _s=s, _d=d, _v=v, _wv=with_vals):
                sv = _s[pl.ds(i, L)]
                dv = _d[pl.ds(i, L)]
                if _wv:
                    vv = _v[pl.ds(i, L)]
                else:
                    vv = jnp.ones((L,), jnp.float32)
                plsc.addupdate_scatter(table, [sv], vv)
                plsc.addupdate_scatter(table, [dv], vv)

        pltpu.sync_copy(table, out_hbm.at[wid])


_scatter_both = pl.kernel(
    _scatter_body,
    out_type=(jax.ShapeDtypeStruct((NW, N_NODES), jnp.float32),
              jax.ShapeDtypeStruct((NW, N_NODES), jnp.float32)),
    mesh=_MESH,
    compiler_params=_SC_PARAMS,
    scratch_types=[
        pltpu.VMEM((N_NODES,), jnp.float32),
        pltpu.VMEM((CHUNK,), jnp.int32),
        pltpu.VMEM((CHUNK,), jnp.int32),
        pltpu.VMEM((CHUNK,), jnp.float32),
        pltpu.VMEM((CHUNK,), jnp.int32),
        pltpu.VMEM((CHUNK,), jnp.int32),
        pltpu.VMEM((CHUNK,), jnp.float32),
        pltpu.SemaphoreType.DMA,
        pltpu.SemaphoreType.DMA,
    ],
)


def _node1_body(head_ref, bed_ref, ovb_ref, bnd_ref, head_o, neff_o):
    h = head_ref[...]
    b = bed_ref[...]
    ov = ovb_ref[...]
    h = jnp.where(bnd_ref[...] != 0.0, b, h)
    head_o[...] = h
    wp = WATER_DENSITY * GRAVITY * (h - b)
    ne = ov - wp
    ne = jnp.where(ne > ov, ov, ne)
    ne = jnp.where(ne < 10000.0, 10000.0, ne)
    neff_o[...] = ne


def _node1(head2, bed2, ovb2, bnd2):
    return pl.pallas_call(
        _node1_body,
        out_shape=(
            jax.ShapeDtypeStruct((NR, NCL), jnp.float32),
            jax.ShapeDtypeStruct((NR, NCL), jnp.float32),
        ),
    )(head2, bed2, ovb2, bnd2)


def _node2_body(velp_ref, degp_ref, neff_ref, geo_ref, melt_o, cond_o):
    vs = jnp.sum(velp_ref[...], axis=0)
    dg = jnp.sum(degp_ref[...], axis=0)
    sliding = vs / jnp.maximum(dg, 1.0)
    ne = neff_ref[...]
    shear = TILL_FRICTION * ne
    friction = jnp.abs(sliding * shear)
    melt = (geo_ref[...] + friction) / LATENT_HEAT
    melt_o[...] = melt
    cond_o[...] = melt / ICE_DENSITY / (ICE_FLUIDITY * (ne * ne * ne))


def _node2(velp, degp, neff2, geo2):
    return pl.pallas_call(
        _node2_body,
        out_shape=(
            jax.ShapeDtypeStruct((NR, NCL), jnp.float32),
            jax.ShapeDtypeStruct((NR, NCL), jnp.float32),
        ),
    )(velp, degp, neff2, geo2)


def _fp_body(cal_ref, grad_ref, re_ref, re_o, tr_o, di_o):
    c = cal_ref[...]
    num = c * c * c * GRAVITY
    g = grad_ref[...]
    r = re_ref[...]
    k = jnp.abs(num * g) * (1.0 / (24.0 * WATER_VISCOSITY * WATER_VISCOSITY))
    for _ in range(N_FP_ITERS):
        r = 0.5 * r + k / (1.0 + FLOW_REGIME_SCALAR * r)
    t = num / (12.0 * WATER_VISCOSITY * (1.0 + FLOW_REGIME_SCALAR * r))
    re_o[...] = r
    tr_o[...] = t
    di_o[...] = -t * g


_EROWS = N_EDGES // ECL
_ROWS_A = E_SPLIT // ECL
_BROW = 1000


def _fp_full(cal, grad, re):
    bspec = pl.BlockSpec((_BROW, ECL), lambda i: (i, 0))
    shape = jax.ShapeDtypeStruct((_EROWS, ECL), jnp.float32)
    return pl.pallas_call(
        _fp_body,
        grid=(_EROWS // _BROW,),
        in_specs=[bspec, bspec, bspec],
        out_specs=(bspec, bspec, bspec),
        out_shape=(shape, shape, shape),
    )(cal.reshape(_EROWS, ECL), grad, re)


def _fp_a(cal_a, grad, re):
    bspec = pl.BlockSpec((_BROW, ECL), lambda i: (i, 0))
    shape = jax.ShapeDtypeStruct((_EROWS, ECL), jnp.float32)
    return pl.pallas_call(
        _fp_body,
        grid=(_ROWS_A // _BROW,),
        in_specs=[bspec, bspec, bspec],
        out_specs=(bspec, bspec, bspec),
        out_shape=(shape, shape, shape),
    )(cal_a.reshape(_ROWS_A, ECL), grad, re)


def _fp_b_body(cal_ref, grad_ref, re_ref, _a, _b, _c, re_o, tr_o, di_o):
    _fp_body(cal_ref, grad_ref, re_ref, re_o, tr_o, di_o)


def _fp_b(cal_b, grad, re, re_f, tr_f, di_f):
    rows_b = _EROWS - _ROWS_A
    near = pl.BlockSpec((_BROW, ECL), lambda i: (i, 0))
    far = pl.BlockSpec((_BROW, ECL), lambda i: (i + _ROWS_A // _BROW, 0))
    anyspec = pl.BlockSpec(memory_space=pl.ANY)
    shape = jax.ShapeDtypeStruct((_EROWS, ECL), jnp.float32)
    return pl.pallas_call(
        _fp_b_body,
        grid=(rows_b // _BROW,),
        in_specs=[near, far, far, anyspec, anyspec, anyspec],
        out_specs=(far, far, far),
        out_shape=(shape, shape, shape),
        input_output_aliases={3: 0, 4: 1, 5: 2},
    )(cal_b.reshape(rows_b, ECL), grad, re, re_f, tr_f, di_f)


def kernel(head, Re, edge_index, bedrock_elevation, overburden_pressure,
           geothermal_heat_flux, ice_sliding_velocity, node_is_boundary):
    src = edge_index[0]
    dst = edge_index[1]
    bnd2 = node_is_boundary.astype(jnp.float32).reshape(NR, NCL)

    velp, degp = _scatter_both(src, dst, ice_sliding_velocity)

    head_p2, neff2 = _node1(
        head.reshape(NR, NCL),
        bedrock_elevation.reshape(NR, NCL),
        overburden_pressure.reshape(NR, NCL),
        bnd2,
    )
    head_p = head_p2.reshape(-1)

    grad = _gather_grad(head_p, src, dst)

    melt2, cond2 = _node2(
        velp.reshape(NW, NR, NCL),
        degp.reshape(NW, NR, NCL),
        neff2,
        geothermal_heat_flux.reshape(NR, NCL),
    )
    cond = cond2.reshape(-1)

    grad2 = grad.reshape(_EROWS, ECL)
    re2 = Re.reshape(_EROWS, ECL)
    cal_a = _gather_mean_a(cond, src, dst)
    cal_b = _gather_mean_b(cond, src, dst)
    re_a, tr_a, di_a = _fp_a(cal_a, grad2, re2)
    re_o, tr_o, di_o = _fp_b(cal_b, grad2, re2, re_a, tr_a, di_a)

    return (
        head_p,
        grad,
        neff2.reshape(-1),
        melt2.reshape(-1),
        cond,
        re_o.reshape(-1),
        tr_o.reshape(-1),
        di_o.reshape(-1),
    )


# speedup vs baseline: 1.0236x; 1.0003x over previous
import jax
import jax.numpy as jnp
from jax import lax
from jax.experimental import pallas as pl
from jax.experimental.pallas import tpu as pltpu
from jax.experimental.pallas import tpu_sc as plsc

N_NODES = 100000
N_EDGES = 3200000
LINK_LENGTH = 100.0
GRAVITY = 9.81
WATER_DENSITY = 1000.0
ICE_DENSITY = 917.0
LATENT_HEAT = 334000.0
WATER_VISCOSITY = 1.787e-06
ICE_FLUIDITY = 6e-24
TILL_FRICTION = 0.5
FLOW_REGIME_SCALAR = 0.001
N_FP_ITERS = 15

NC, NS, L = 2, 16, 16
NW = NC * NS
EPW = N_EDGES // NW
CHUNK = 4000
NCHUNKS = EPW // CHUNK
UNROLL = 5

E_SPLIT = 1664000

_MESH = plsc.VectorSubcoreMesh(
    core_axis_name="c", subcore_axis_name="s", num_cores=NC, num_subcores=NS)
_SC_PARAMS = pltpu.CompilerParams(
    needs_layout_passes=False, use_tc_tiling_on_sc=False)

NR, NCL = 100, 1000
ECL = 128


def _worker_id():
    return lax.axis_index("s") * NC + lax.axis_index("c")


def _make_gather(mode, estart, ecount):
    epw = ecount // NW
    nchunks = epw // CHUNK

    def body(tab_hbm, src_hbm, dst_hbm, out_hbm, table, shared,
             srcv0, dstv0, outv0, srcv1, dstv1, outv1,
             tsem, isem0, isem1, osem0, osem1):
        sid = lax.axis_index("s")
        obase = _worker_id() * epw
        base = estart + obase
        bufs = ((srcv0, dstv0, outv0, isem0, osem0),
                (srcv1, dstv1, outv1, isem1, osem1))

        def start_in(ci):
            s, d, _, isem, _ = bufs[ci % 2]
            off = base + ci * CHUNK
            c1 = pltpu.async_copy(src_hbm.at[pl.ds(off, CHUNK)], s, isem)
            c2 = pltpu.async_copy(dst_hbm.at[pl.ds(off, CHUNK)], d, isem)
            return (c1, c2)

        in_cp = {0: start_in(0), 1: start_in(1)}

        @pl.when(sid == 0)
        def _():
            pltpu.sync_copy(tab_hbm, shared)

        plsc.subcore_barrier()
        table_cp = pltpu.async_copy(shared, table, tsem)

        out_cp = {}
        for ci in range(nchunks):
            s, d, o, isem, osem = bufs[ci % 2]
            if ci + 1 < nchunks and ci > 0:
                in_cp[ci + 1] = start_in(ci + 1)
            for cp in in_cp.pop(ci):
                cp.wait()
            if ci == 0:
                table_cp.wait()
            if ci >= 2:
                out_cp.pop(ci - 2).wait()

            @plsc.parallel_loop(0, CHUNK, step=L, unroll=UNROLL)
            def _(i, _s=s, _d=d, _o=o):
                sv = _s[pl.ds(i, L)]
                dv = _d[pl.ds(i, L)]
                ts = plsc.load_gather(table, [sv])
                td = plsc.load_gather(table, [dv])
                if mode == 0:
                    _o[pl.ds(i, L)] = (td - ts) / LINK_LENGTH
                else:
                    _o[pl.ds(i, L)] = 0.5 * (ts + td)

            out_cp[ci] = pltpu.async_copy(
                o, out_hbm.at[pl.ds(obase + ci * CHUNK, CHUNK)], osem)
        for cp in out_cp.values():
            cp.wait()

    return pl.kernel(
        body,
        out_type=jax.ShapeDtypeStruct((ecount,), jnp.float32),
        mesh=_MESH,
        compiler_params=_SC_PARAMS,
        scratch_types=[
            pltpu.VMEM((N_NODES,), jnp.float32),
            pltpu.VMEM_SHARED((N_NODES,), jnp.float32),
            pltpu.VMEM((CHUNK,), jnp.int32),
            pltpu.VMEM((CHUNK,), jnp.int32),
            pltpu.VMEM((CHUNK,), jnp.float32),
            pltpu.VMEM((CHUNK,), jnp.int32),
            pltpu.VMEM((CHUNK,), jnp.int32),
            pltpu.VMEM((CHUNK,), jnp.float32),
            pltpu.SemaphoreType.DMA,
            pltpu.SemaphoreType.DMA,
            pltpu.SemaphoreType.DMA,
            pltpu.SemaphoreType.DMA,
            pltpu.SemaphoreType.DMA,
        ],
    )


_gather_grad = _make_gather(0, 0, N_EDGES)
_gather_mean_a = _make_gather(1, 0, E_SPLIT)
_gather_mean_b = _make_gather(1, E_SPLIT, N_EDGES - E_SPLIT)


def _scatter_body(src_hbm, dst_hbm, val_hbm, velp_hbm, degp_hbm, table,
                  srcv0, dstv0, valv0, srcv1, dstv1, valv1, isem0, isem1):
    wid = _worker_id()
    base = wid * EPW
    bufs = ((srcv0, dstv0, valv0, isem0),
            (srcv1, dstv1, valv1, isem1))

    def start_in(ci, with_vals):
        s, d, v, isem = bufs[ci % 2]
        off = base + ci * CHUNK
        cps = [pltpu.async_copy(src_hbm.at[pl.ds(off, CHUNK)], s, isem),
               pltpu.async_copy(dst_hbm.at[pl.ds(off, CHUNK)], d, isem)]
        if with_vals:
            cps.append(
                pltpu.async_copy(val_hbm.at[pl.ds(off, CHUNK)], v, isem))
        return cps

    for phase, out_hbm in ((0, velp_hbm), (1, degp_hbm)):
        with_vals = phase == 0
        in_cp = {0: start_in(0, with_vals)}

        @plsc.parallel_loop(0, N_NODES, step=L, unroll=25)
        def _(i):
            table[pl.ds(i, L)] = jnp.zeros((L,), jnp.float32)

        for ci in range(NCHUNKS):
            s, d, v, isem = bufs[ci % 2]
            if ci + 1 < NCHUNKS:
                in_cp[ci + 1] = start_in(ci + 1, with_vals)
            for cp in in_cp.pop(ci):
                cp.wait()

            @plsc.parallel_loop(0, CHUNK, step=L, unroll=UNROLL)
            def _(i, _s=s, _d=d, _v=v, _wv=with_vals):
                sv = _s[pl.ds(i, L)]
                dv = _d[pl.ds(i, L)]
                if _wv:
                    vv = _v[pl.ds(i, L)]
                else:
                    vv = jnp.ones((L,), jnp.float32)
                plsc.addupdate_scatter(table, [sv], vv)
                plsc.addupdate_scatter(table, [dv], vv)

        pltpu.sync_copy(table, out_hbm.at[wid])


_scatter_both = pl.kernel(
    _scatter_body,
    out_type=(jax.ShapeDtypeStruct((NW, N_NODES), jnp.float32),
              jax.ShapeDtypeStruct((NW, N_NODES), jnp.float32)),
    mesh=_MESH,
    compiler_params=_SC_PARAMS,
    scratch_types=[
        pltpu.VMEM((N_NODES,), jnp.float32),
        pltpu.VMEM((CHUNK,), jnp.int32),
        pltpu.VMEM((CHUNK,), jnp.int32),
        pltpu.VMEM((CHUNK,), jnp.float32),
        pltpu.VMEM((CHUNK,), jnp.int32),
        pltpu.VMEM((CHUNK,), jnp.int32),
        pltpu.VMEM((CHUNK,), jnp.float32),
        pltpu.SemaphoreType.DMA,
        pltpu.SemaphoreType.DMA,
    ],
)


def _node1_body(head_ref, bed_ref, ovb_ref, bnd_ref, head_o, neff_o):
    h = head_ref[...]
    b = bed_ref[...]
    ov = ovb_ref[...]
    h = jnp.where(bnd_ref[...] != 0.0, b, h)
    head_o[...] = h
    wp = WATER_DENSITY * GRAVITY * (h - b)
    ne = ov - wp
    ne = jnp.where(ne > ov, ov, ne)
    ne = jnp.where(ne < 10000.0, 10000.0, ne)
    neff_o[...] = ne


def _node1(head2, bed2, ovb2, bnd2):
    return pl.pallas_call(
        _node1_body,
        out_shape=(
            jax.ShapeDtypeStruct((NR, NCL), jnp.float32),
            jax.ShapeDtypeStruct((NR, NCL), jnp.float32),
        ),
    )(head2, bed2, ovb2, bnd2)


def _node2_body(velp_ref, degp_ref, neff_ref, geo_ref, melt_o, cond_o):
    vs = jnp.sum(velp_ref[...], axis=0)
    dg = jnp.sum(degp_ref[...], axis=0)
    sliding = vs / jnp.maximum(dg, 1.0)
    ne = neff_ref[...]
    shear = TILL_FRICTION * ne
    friction = jnp.abs(sliding * shear)
    melt = (geo_ref[...] + friction) / LATENT_HEAT
    melt_o[...] = melt
    cond_o[...] = melt / ICE_DENSITY / (ICE_FLUIDITY * (ne * ne * ne))


def _node2(velp, degp, neff2, geo2):
    return pl.pallas_call(
        _node2_body,
        out_shape=(
            jax.ShapeDtypeStruct((NR, NCL), jnp.float32),
            jax.ShapeDtypeStruct((NR, NCL), jnp.float32),
        ),
    )(velp, degp, neff2, geo2)


def _fp_body(cal_ref, grad_ref, re_ref, re_o, tr_o, di_o):
    c = cal_ref[...]
    num = c * c * c * GRAVITY
    g = grad_ref[...]
    r = re_ref[...]
    k = jnp.abs(num * g) * (1.0 / (24.0 * WATER_VISCOSITY * WATER_VISCOSITY))
    for _ in range(N_FP_ITERS):
        r = 0.5 * r + k / (1.0 + FLOW_REGIME_SCALAR * r)
    t = num / (12.0 * WATER_VISCOSITY * (1.0 + FLOW_REGIME_SCALAR * r))
    re_o[...] = r
    tr_o[...] = t
    di_o[...] = -t * g


_EROWS = N_EDGES // ECL
_ROWS_A = E_SPLIT // ECL
_BROW = 1000


def _fp_a(cal_a, grad, re):
    bspec = pl.BlockSpec((_BROW, ECL), lambda i: (i, 0))
    shape = jax.ShapeDtypeStruct((_EROWS, ECL), jnp.float32)
    return pl.pallas_call(
        _fp_body,
        grid=(_ROWS_A // _BROW,),
        in_specs=[bspec, bspec, bspec],
        out_specs=(bspec, bspec, bspec),
        out_shape=(shape, shape, shape),
    )(cal_a.reshape(_ROWS_A, ECL), grad, re)


def _fp_b_body(cal_ref, grad_ref, re_ref, _a, _b, _c, re_o, tr_o, di_o):
    _fp_body(cal_ref, grad_ref, re_ref, re_o, tr_o, di_o)


def _fp_b(cal_b, grad, re, re_f, tr_f, di_f):
    rows_b = _EROWS - _ROWS_A
    near = pl.BlockSpec((_BROW, ECL), lambda i: (i, 0))
    far = pl.BlockSpec((_BROW, ECL), lambda i: (i + _ROWS_A // _BROW, 0))
    anyspec = pl.BlockSpec(memory_space=pl.ANY)
    shape = jax.ShapeDtypeStruct((_EROWS, ECL), jnp.float32)
    return pl.pallas_call(
        _fp_b_body,
        grid=(rows_b // _BROW,),
        in_specs=[near, far, far, anyspec, anyspec, anyspec],
        out_specs=(far, far, far),
        out_shape=(shape, shape, shape),
        input_output_aliases={3: 0, 4: 1, 5: 2},
    )(cal_b.reshape(rows_b, ECL), grad, re, re_f, tr_f, di_f)


def kernel(head, Re, edge_index, bedrock_elevation, overburden_pressure,
           geothermal_heat_flux, ice_sliding_velocity, node_is_boundary):
    src = edge_index[0]
    dst = edge_index[1]
    bnd2 = node_is_boundary.astype(jnp.float32).reshape(NR, NCL)

    velp, degp = _scatter_both(src, dst, ice_sliding_velocity)

    head_p2, neff2 = _node1(
        head.reshape(NR, NCL),
        bedrock_elevation.reshape(NR, NCL),
        overburden_pressure.reshape(NR, NCL),
        bnd2,
    )
    head_p = head_p2.reshape(-1)

    grad = _gather_grad(head_p, src, dst)

    melt2, cond2 = _node2(
        velp.reshape(NW, NR, NCL),
        degp.reshape(NW, NR, NCL),
        neff2,
        geothermal_heat_flux.reshape(NR, NCL),
    )
    cond = cond2.reshape(-1)

    grad2 = grad.reshape(_EROWS, ECL)
    re2 = Re.reshape(_EROWS, ECL)
    cal_a = _gather_mean_a(cond, src, dst)
    cal_b = _gather_mean_b(cond, src, dst)
    re_a, tr_a, di_a = _fp_a(cal_a, grad2, re2)
    re_o, tr_o, di_o = _fp_b(cal_b, grad2, re2, re_a, tr_a, di_a)

    return (
        head_p,
        grad,
        neff2.reshape(-1),
        melt2.reshape(-1),
        cond,
        re_o.reshape(-1),
        tr_o.reshape(-1),
        di_o.reshape(-1),
    )
